# baseline (speedup 1.0000x reference)
import functools

import jax
import jax.numpy as jnp
from jax import lax
from jax.experimental import pallas as pl
from jax.experimental.pallas import tpu as pltpu
from jax.experimental.pallas import tpu_sc as plsc

D = 64
B_ROWS = 4096
SEQ = 200

_info = plsc.get_sparse_core_info()
NC, NS = _info.num_cores, _info.num_subcores
NW = NC * NS
TR = B_ROWS // NW
NBUF = 4
S = 2
N_STEPS = TR + S
N_OUTER = (N_STEPS + NBUF - 1) // NBUF

_mesh = plsc.VectorSubcoreMesh(core_axis_name="c", subcore_axis_name="s")


@functools.partial(
    pl.kernel,
    mesh=_mesh,
    out_type=jax.ShapeDtypeStruct((B_ROWS, SEQ, D), jnp.float32),
    compiler_params=pltpu.CompilerParams(use_tc_tiling_on_sc=False),
    scratch_types=[
        pltpu.VMEM((TR, SEQ), jnp.int32),
        pltpu.VMEM((SEQ, D), jnp.float32),
        pltpu.VMEM((SEQ, D), jnp.float32),
        pltpu.VMEM((SEQ, D), jnp.float32),
        pltpu.VMEM((SEQ, D), jnp.float32),
        pltpu.SemaphoreType.DMA,
        pltpu.SemaphoreType.DMA,
        pltpu.SemaphoreType.DMA,
        pltpu.SemaphoreType.DMA,
        pltpu.SemaphoreType.DMA,
        pltpu.SemaphoreType.DMA,
        pltpu.SemaphoreType.DMA,
        pltpu.SemaphoreType.DMA,
    ],
)
def _gather_kernel(tok_hbm, table_hbm, out_hbm, idx_v,
                   rows0, rows1, rows2, rows3,
                   gsem0, gsem1, gsem2, gsem3,
                   wsem0, wsem1, wsem2, wsem3):
    rows = [rows0, rows1, rows2, rows3]
    gsem = [gsem0, gsem1, gsem2, gsem3]
    wsem = [wsem0, wsem1, wsem2, wsem3]

    wid = lax.axis_index("s") * NC + lax.axis_index("c")
    rbase = wid * TR

    pltpu.sync_copy(tok_hbm.at[pl.ds(rbase, TR)], idx_v)

    def gather_copy(g, b):
        src = table_hbm.at[idx_v.at[g]]
        return pltpu.make_async_copy(src, rows[b], gsem[b])

    def wb_copy(g, b):
        dst = out_hbm.at[rbase + g]
        return pltpu.make_async_copy(rows[b], dst, wsem[b])

    def outer(g0, carry):
        for j in range(NBUF):
            g = g0 * NBUF + j
            bc = (j + NBUF - S) % NBUF

            @pl.when(g >= NBUF)
            def _():
                wb_copy(g - NBUF, j).wait()

            @pl.when(g < TR)
            def _():
                gather_copy(g, j).start()

            @pl.when(jnp.logical_and(g >= S, g < TR + S))
            def _():
                gather_copy(g - S, bc).wait()
                wb_copy(g - S, bc).start()
        return carry

    lax.fori_loop(0, N_OUTER, outer, 0)


def kernel(token_ids, embedding):
    return _gather_kernel(token_ids.astype(jnp.int32), embedding)
